# baseline (speedup 1.0000x reference)
import functools

import jax
import jax.numpy as jnp
from jax import lax
from jax.experimental import pallas as pl
from jax.experimental.pallas import tpu as pltpu
from jax.experimental.pallas import tpu_sc as plsc

OMEGA = 32
SINK = 4
P_RATIO = 0.1
R_RATIO = 0.3
START_IDX = 1

W_PAD = 64
LANES = 16


def _fused_body(h_num, s_len, nq, idx_pad, n_eligible, k_windows, sink,
                omega, mid_end, local_off, kept_len, attn_ref, idx_ref,
                acc_ref):
    h = pl.program_id(0)
    q = pl.program_id(1)
    part = attn_ref[0].sum(axis=0)

    @pl.when(q == 0)
    def _init():
        acc_ref[h, :] = part

    @pl.when(q > 0)
    def _acc():
        acc_ref[h, :] += part

    @pl.when((h == h_num - 1) & (q == nq - 1))
    def _tail():
        _index_tail(h_num, s_len, idx_pad, n_eligible, k_windows, sink,
                    omega, mid_end, local_off, kept_len, acc_ref, idx_ref)


def _build_fused_kernel(h_num, s_len, idx_pad, n_eligible, k_windows,
                        mid_end, local_off, kept_len):
    qc = 1024
    nq = s_len // qc
    body = functools.partial(_fused_body, h_num, s_len, nq, idx_pad,
                             n_eligible, k_windows, SINK, OMEGA, mid_end,
                             local_off, kept_len)
    return pl.pallas_call(
        body,
        grid=(h_num, nq),
        in_specs=[pl.BlockSpec((1, qc, s_len), lambda h, q: (h, q, 0))],
        out_specs=pl.BlockSpec((h_num, 1, idx_pad), lambda h, q: (0, 0, 0)),
        out_shape=jax.ShapeDtypeStruct((h_num, 1, idx_pad), jnp.int32),
        scratch_shapes=[pltpu.VMEM((h_num, s_len), jnp.float32)],
        compiler_params=pltpu.CompilerParams(
            dimension_semantics=("arbitrary", "arbitrary")),
    )


def _index_tail(h_num, s_len, idx_pad, n_eligible, k_windows, sink, omega,
                mid_end, local_off, kept_len, col_ref, idx_ref):
    cs = col_ref[...].reshape(h_num, s_len)

    w3 = lax.broadcasted_iota(jnp.int32, (h_num, W_PAD, s_len), 1)
    s3 = lax.broadcasted_iota(jnp.int32, (h_num, W_PAD, s_len), 2)
    in_win = (s3 >= sink) & ((s3 - sink) // omega == w3) & (w3 < n_eligible)
    cs3 = jnp.broadcast_to(cs[:, None, :], (h_num, W_PAD, s_len))
    win = jnp.where(in_win, cs3, 0.0).sum(axis=2)

    l64 = lax.broadcasted_iota(jnp.int32, (h_num, W_PAD), 1)
    neg = jnp.float32(-jnp.inf)
    base = jnp.where(l64 < n_eligible, win, neg)

    def step(_, keep):
        cur = jnp.where(keep > 0, neg, base)
        m = jnp.max(cur, axis=1, keepdims=True)
        first = jnp.min(jnp.where(cur == m, l64, W_PAD), axis=1, keepdims=True)
        return jnp.where(l64 == first, 1, keep)

    km_i = lax.fori_loop(0, k_windows, step,
                         jnp.zeros((h_num, W_PAD), jnp.int32))
    km = km_i > 0

    wr = lax.broadcasted_iota(jnp.int32, (W_PAD, W_PAD), 0)
    wp = lax.broadcasted_iota(jnp.int32, (W_PAD, W_PAD), 1)
    le = (wr <= wp).astype(jnp.float32)
    cums = jax.lax.dot_general(km.astype(jnp.float32), le,
                               (((1,), (0,)), ((), ())),
                               precision=jax.lax.Precision.HIGHEST)
    pos = cums.astype(jnp.int32) - 1

    sl = lax.broadcasted_iota(jnp.int32, (1, idx_pad), 1)
    jm = (sl - sink) // omega
    rm = (sl - sink) % omega
    tok_mid = jnp.zeros((h_num, idx_pad), jnp.int32)
    for j in range(k_windows):
        pj = jnp.where(km & (pos == j), l64, 0).sum(axis=1, keepdims=True)
        tok_mid = tok_mid + jnp.where(jm == j, pj * omega, 0)
    tok = jnp.where(sl < sink, sl,
                    jnp.where(sl >= mid_end, sl + local_off,
                              tok_mid + sink + rm))
    tok = jnp.where(sl >= kept_len, 0, tok)
    idx_ref[...] = tok[:, None, :]


def _build_sc_colgather(h_num, d, s_len, kept, idx_pad):
    mesh = plsc.VectorSubcoreMesh(core_axis_name="c", subcore_axis_name="s")
    dq = LANES
    nq = d // dq
    ngrp = kept // LANES
    w_main = (kept // 128) * 128
    w_tail = kept - w_main

    @functools.partial(
        pl.kernel, mesh=mesh,
        out_type=(jax.ShapeDtypeStruct((h_num, d, kept), jnp.float32),
                  jax.ShapeDtypeStruct((h_num, d, kept), jnp.float32)),
        scratch_types=[pltpu.VMEM((idx_pad,), jnp.int32),
                       pltpu.VMEM((2 * dq * s_len,), jnp.float32),
                       pltpu.VMEM((d, kept), jnp.float32),
                       pltpu.SemaphoreType.DMA,
                       pltpu.SemaphoreType.DMA],
        compiler_params=pltpu.CompilerParams(needs_layout_passes=False),
    )
    def gat(keys_hbm, vals_hbm, idx_hbm, out_k, out_v, idxb, qbuf, obuf,
            sem, semw):
        wid = lax.axis_index("s") * 2 + lax.axis_index("c")
        head = wid % h_num

        def stage(tab, q):
            slot = (q % 2) * dq * s_len
            for dl in range(dq):
                pltpu.async_copy(tab.at[head, q * dq + dl],
                                 qbuf.at[pl.ds(slot + dl * s_len, s_len)], sem)

        def stage_wait(tab, q):
            slot = (q % 2) * dq * s_len
            for dl in range(dq):
                pltpu.make_async_copy(
                    tab.at[head, q * dq + dl],
                    qbuf.at[pl.ds(slot + dl * s_len, s_len)], sem).wait()

        def compact(tab, out):
            pltpu.sync_copy(idx_hbm.at[head, 0], idxb)
            stage(tab, 0)
            for q in range(nq):
                stage_wait(tab, q)
                if q + 1 < nq:
                    stage(tab, q + 1)
                slot = (q % 2) * dq * s_len

                def jgrp(j0, carry):
                    colv = idxb[pl.ds(j0 * LANES, LANES)]
                    for dl in range(dq):
                        vals = plsc.load_gather(
                            qbuf, [colv + (slot + dl * s_len)])
                        obuf[q * dq + dl, pl.ds(j0 * LANES, LANES)] = vals
                    return carry

                lax.fori_loop(0, ngrp, jgrp, 0)
                pltpu.async_copy(obuf.at[pl.ds(q * dq, dq), pl.ds(0, w_main)],
                                 out.at[head, pl.ds(q * dq, dq),
                                        pl.ds(0, w_main)], semw)
                if w_tail:
                    pltpu.async_copy(
                        obuf.at[pl.ds(q * dq, dq), pl.ds(w_main, w_tail)],
                        out.at[head, pl.ds(q * dq, dq),
                               pl.ds(w_main, w_tail)], semw)
            for q in range(nq):
                pltpu.make_async_copy(
                    obuf.at[pl.ds(q * dq, dq), pl.ds(0, w_main)],
                    out.at[head, pl.ds(q * dq, dq), pl.ds(0, w_main)],
                    semw).wait()
                if w_tail:
                    pltpu.make_async_copy(
                        obuf.at[pl.ds(q * dq, dq), pl.ds(w_main, w_tail)],
                        out.at[head, pl.ds(q * dq, dq),
                               pl.ds(w_main, w_tail)], semw).wait()

        @pl.when(wid < h_num)
        def _k():
            compact(keys_hbm, out_k)

        @pl.when(wid >= h_num)
        def _v():
            compact(vals_hbm, out_v)

    return gat


def kernel(past_key, past_value, attn_score_cache):
    b, h_num, s_len, d = past_key.shape
    assert b == 1
    local_num = int(P_RATIO * s_len) // OMEGA
    n_win = (s_len - SINK) // OMEGA
    budget_tokens = int(R_RATIO * s_len)
    k_windows = max((budget_tokens - SINK) // OMEGA - 1 - local_num - START_IDX, 1)
    n_eligible = n_win - local_num
    local_start = SINK + n_eligible * OMEGA
    mid_end = SINK + k_windows * OMEGA
    kept_len = mid_end + (s_len - local_start)
    local_off = local_start - mid_end
    assert n_win <= W_PAD
    idx_pad = -(-kept_len // 128) * 128

    attn3 = attn_score_cache.reshape(h_num, s_len, s_len)
    idx = _build_fused_kernel(h_num, s_len, idx_pad, n_eligible, k_windows,
                              mid_end, local_off, kept_len)(attn3)

    ktr = jnp.transpose(past_key, (0, 1, 3, 2)).reshape(h_num, d, s_len)
    vtr = jnp.transpose(past_value, (0, 1, 3, 2)).reshape(h_num, d, s_len)
    okt, ovt = _build_sc_colgather(h_num, d, s_len, kept_len, idx_pad)(
        ktr, vtr, idx)
    new_k = jnp.transpose(okt.reshape(b, h_num, d, kept_len), (0, 1, 3, 2))
    new_v = jnp.transpose(ovt.reshape(b, h_num, d, kept_len), (0, 1, 3, 2))
    return new_k, new_v


# speedup vs baseline: 1.0226x; 1.0226x over previous
import functools

import jax
import jax.numpy as jnp
from jax import lax
from jax.experimental import pallas as pl
from jax.experimental.pallas import tpu as pltpu
from jax.experimental.pallas import tpu_sc as plsc

OMEGA = 32
SINK = 4
P_RATIO = 0.1
R_RATIO = 0.3
START_IDX = 1

W_PAD = 64
LANES = 16


def _fused_body(h_num, s_len, idx_pad, n_eligible, k_windows, sink, omega,
                mid_end, local_off, kept_len, attn_ref, idx_ref, acc_ref):
    h = pl.program_id(0)
    acc_ref[h, :] = attn_ref[0].sum(axis=0)

    @pl.when(h == h_num - 1)
    def _tail():
        _index_tail(h_num, s_len, idx_pad, n_eligible, k_windows, sink,
                    omega, mid_end, local_off, kept_len, acc_ref, idx_ref)


def _build_fused_kernel(h_num, s_len, idx_pad, n_eligible, k_windows,
                        mid_end, local_off, kept_len):
    body = functools.partial(_fused_body, h_num, s_len, idx_pad, n_eligible,
                             k_windows, SINK, OMEGA, mid_end, local_off,
                             kept_len)
    return pl.pallas_call(
        body,
        grid=(h_num,),
        in_specs=[pl.BlockSpec((1, s_len, s_len), lambda h: (h, 0, 0))],
        out_specs=pl.BlockSpec((h_num, 1, idx_pad), lambda h: (0, 0, 0)),
        out_shape=jax.ShapeDtypeStruct((h_num, 1, idx_pad), jnp.int32),
        scratch_shapes=[pltpu.VMEM((h_num, s_len), jnp.float32)],
        compiler_params=pltpu.CompilerParams(
            dimension_semantics=("arbitrary",)),
    )


def _index_tail(h_num, s_len, idx_pad, n_eligible, k_windows, sink, omega,
                mid_end, local_off, kept_len, col_ref, idx_ref):
    cs = col_ref[...].reshape(h_num, s_len)

    w3 = lax.broadcasted_iota(jnp.int32, (h_num, W_PAD, s_len), 1)
    s3 = lax.broadcasted_iota(jnp.int32, (h_num, W_PAD, s_len), 2)
    in_win = (s3 >= sink) & ((s3 - sink) // omega == w3) & (w3 < n_eligible)
    cs3 = jnp.broadcast_to(cs[:, None, :], (h_num, W_PAD, s_len))
    win = jnp.where(in_win, cs3, 0.0).sum(axis=2)

    l64 = lax.broadcasted_iota(jnp.int32, (h_num, W_PAD), 1)
    neg = jnp.float32(-jnp.inf)
    base = jnp.where(l64 < n_eligible, win, neg)

    def step(_, keep):
        cur = jnp.where(keep > 0, neg, base)
        m = jnp.max(cur, axis=1, keepdims=True)
        first = jnp.min(jnp.where(cur == m, l64, W_PAD), axis=1, keepdims=True)
        return jnp.where(l64 == first, 1, keep)

    km_i = lax.fori_loop(0, k_windows, step,
                         jnp.zeros((h_num, W_PAD), jnp.int32))
    km = km_i > 0

    wr = lax.broadcasted_iota(jnp.int32, (W_PAD, W_PAD), 0)
    wp = lax.broadcasted_iota(jnp.int32, (W_PAD, W_PAD), 1)
    le = (wr <= wp).astype(jnp.float32)
    cums = jax.lax.dot_general(km.astype(jnp.float32), le,
                               (((1,), (0,)), ((), ())),
                               precision=jax.lax.Precision.HIGHEST)
    pos = cums.astype(jnp.int32) - 1

    sl = lax.broadcasted_iota(jnp.int32, (1, idx_pad), 1)
    jm = (sl - sink) // omega
    rm = (sl - sink) % omega
    tok_mid = jnp.zeros((h_num, idx_pad), jnp.int32)
    for j in range(k_windows):
        pj = jnp.where(km & (pos == j), l64, 0).sum(axis=1, keepdims=True)
        tok_mid = tok_mid + jnp.where(jm == j, pj * omega, 0)
    tok = jnp.where(sl < sink, sl,
                    jnp.where(sl >= mid_end, sl + local_off,
                              tok_mid + sink + rm))
    tok = jnp.where(sl >= kept_len, 0, tok)
    idx_ref[...] = tok[:, None, :]


def _build_sc_colgather(h_num, d, s_len, kept, idx_pad):
    mesh = plsc.VectorSubcoreMesh(core_axis_name="c", subcore_axis_name="s")
    dq = LANES
    nq = d // dq
    ngrp = kept // LANES
    w_main = (kept // 128) * 128
    w_tail = kept - w_main

    @functools.partial(
        pl.kernel, mesh=mesh,
        out_type=(jax.ShapeDtypeStruct((h_num, d, kept), jnp.float32),
                  jax.ShapeDtypeStruct((h_num, d, kept), jnp.float32)),
        scratch_types=[pltpu.VMEM((idx_pad,), jnp.int32),
                       pltpu.VMEM((2 * dq * s_len,), jnp.float32),
                       pltpu.VMEM((d, kept), jnp.float32),
                       pltpu.SemaphoreType.DMA,
                       pltpu.SemaphoreType.DMA],
        compiler_params=pltpu.CompilerParams(needs_layout_passes=False),
    )
    def gat(keys_hbm, vals_hbm, idx_hbm, out_k, out_v, idxb, qbuf, obuf,
            sem, semw):
        wid = lax.axis_index("s") * 2 + lax.axis_index("c")
        head = wid % h_num

        def stage(tab, q):
            slot = (q % 2) * dq * s_len
            for dl in range(dq):
                pltpu.async_copy(tab.at[head, q * dq + dl],
                                 qbuf.at[pl.ds(slot + dl * s_len, s_len)], sem)

        def stage_wait(tab, q):
            slot = (q % 2) * dq * s_len
            for dl in range(dq):
                pltpu.make_async_copy(
                    tab.at[head, q * dq + dl],
                    qbuf.at[pl.ds(slot + dl * s_len, s_len)], sem).wait()

        def compact(tab, out):
            pltpu.sync_copy(idx_hbm.at[head, 0], idxb)
            stage(tab, 0)
            for q in range(nq):
                stage_wait(tab, q)
                if q + 1 < nq:
                    stage(tab, q + 1)
                slot = (q % 2) * dq * s_len

                def jgrp(j0, carry):
                    colv = idxb[pl.ds(j0 * LANES, LANES)]
                    for dl in range(dq):
                        vals = plsc.load_gather(
                            qbuf, [colv + (slot + dl * s_len)])
                        obuf[q * dq + dl, pl.ds(j0 * LANES, LANES)] = vals
                    return carry

                lax.fori_loop(0, ngrp, jgrp, 0)
                pltpu.async_copy(obuf.at[pl.ds(q * dq, dq), pl.ds(0, w_main)],
                                 out.at[head, pl.ds(q * dq, dq),
                                        pl.ds(0, w_main)], semw)
                if w_tail:
                    pltpu.async_copy(
                        obuf.at[pl.ds(q * dq, dq), pl.ds(w_main, w_tail)],
                        out.at[head, pl.ds(q * dq, dq),
                               pl.ds(w_main, w_tail)], semw)
            for q in range(nq):
                pltpu.make_async_copy(
                    obuf.at[pl.ds(q * dq, dq), pl.ds(0, w_main)],
                    out.at[head, pl.ds(q * dq, dq), pl.ds(0, w_main)],
                    semw).wait()
                if w_tail:
                    pltpu.make_async_copy(
                        obuf.at[pl.ds(q * dq, dq), pl.ds(w_main, w_tail)],
                        out.at[head, pl.ds(q * dq, dq),
                               pl.ds(w_main, w_tail)], semw).wait()

        @pl.when(wid < h_num)
        def _k():
            compact(keys_hbm, out_k)

        @pl.when(wid >= h_num)
        def _v():
            compact(vals_hbm, out_v)

    return gat


def kernel(past_key, past_value, attn_score_cache):
    b, h_num, s_len, d = past_key.shape
    assert b == 1
    local_num = int(P_RATIO * s_len) // OMEGA
    n_win = (s_len - SINK) // OMEGA
    budget_tokens = int(R_RATIO * s_len)
    k_windows = max((budget_tokens - SINK) // OMEGA - 1 - local_num - START_IDX, 1)
    n_eligible = n_win - local_num
    local_start = SINK + n_eligible * OMEGA
    mid_end = SINK + k_windows * OMEGA
    kept_len = mid_end + (s_len - local_start)
    local_off = local_start - mid_end
    assert n_win <= W_PAD
    idx_pad = -(-kept_len // 128) * 128

    attn3 = attn_score_cache.reshape(h_num, s_len, s_len)
    idx = _build_fused_kernel(h_num, s_len, idx_pad, n_eligible, k_windows,
                              mid_end, local_off, kept_len)(attn3)

    ktr = jnp.transpose(past_key, (0, 1, 3, 2)).reshape(h_num, d, s_len)
    vtr = jnp.transpose(past_value, (0, 1, 3, 2)).reshape(h_num, d, s_len)
    okt, ovt = _build_sc_colgather(h_num, d, s_len, kept_len, idx_pad)(
        ktr, vtr, idx)
    new_k = jnp.transpose(okt.reshape(b, h_num, d, kept_len), (0, 1, 3, 2))
    new_v = jnp.transpose(ovt.reshape(b, h_num, d, kept_len), (0, 1, 3, 2))
    return new_k, new_v
